# baseline (speedup 1.0000x reference)
import functools

import jax
import jax.numpy as jnp
from jax import lax
from jax.experimental import pallas as pl
from jax.experimental.pallas import tpu as pltpu
from jax.experimental.pallas import tpu_sc as plsc

_B, _L, _D = 16, 4096, 1024
_NC, _NS, _LANES = 2, 16, 16
_NW = _NC * _NS
_RPW = _B * _L // _NW
_CH = 8
_NBUF = 14
_LOOKAHEAD = 6
_NCHUNK = _RPW // _CH


def _sc_body(hid_hbm, mask_hbm, out_hbm, mask_v, lidx_v, bufs_v, sem_g, sem_s):
    wid = lax.axis_index("s") * _NC + lax.axis_index("c")
    b = wid // 2
    half = wid % 2
    lo = half * _RPW
    row0 = b * _L

    pltpu.sync_copy(mask_hbm.at[b], mask_v)

    def init_body(i, _):
        lidx_v[pl.ds(i * _LANES, _LANES)] = jnp.full((_LANES,), row0, jnp.int32)
        return 0

    lax.fori_loop(0, _RPW // _LANES, init_body, 0, unroll=False)

    def scan_body(i, carry):
        m = mask_v[pl.ds(i * _LANES, _LANES)]
        rank = plsc.cumsum(m) + (carry - 1)
        tok = lax.iota(jnp.int32, _LANES) + (i * _LANES + row0)
        valid = (m > 0) & (rank >= lo) & (rank < lo + _RPW)
        local = jnp.clip(rank - lo, 0, _RPW - 1)
        plsc.store_scatter(lidx_v, [local], tok, mask=valid)
        return carry + jnp.sum(m)

    lax.fori_loop(0, _L // _LANES, scan_body, jnp.int32(0), unroll=False)

    out_base = wid * _RPW

    def gather_start(c, p):
        pltpu.async_copy(
            hid_hbm.at[lidx_v.at[pl.ds(c * _CH, _CH)]],
            bufs_v.at[pl.ds(p * _CH, _CH)],
            sem_g,
        )

    def gather_wait():
        pltpu.make_async_copy(
            hid_hbm.at[lidx_v.at[pl.ds(0, _CH)]],
            bufs_v.at[pl.ds(0, _CH)],
            sem_g,
        ).wait()

    def write_start(c, p):
        pltpu.async_copy(
            bufs_v.at[pl.ds(p * _CH, _CH)],
            out_hbm.at[pl.ds(out_base + c * _CH, _CH)],
            sem_s,
        )

    def write_wait():
        pltpu.make_async_copy(
            bufs_v.at[pl.ds(0, _CH)],
            out_hbm.at[pl.ds(out_base, _CH)],
            sem_s,
        ).wait()

    for c0 in range(_LOOKAHEAD):
        gather_start(c0, c0)

    def chunk_body(c, _):
        @pl.when(c + _LOOKAHEAD < _NCHUNK)
        def _():
            @pl.when(c + _LOOKAHEAD >= _NBUF)
            def _():
                write_wait()

            gather_start(c + _LOOKAHEAD, (c + _LOOKAHEAD) % _NBUF)

        gather_wait()
        write_start(c, c % _NBUF)
        return 0

    lax.fori_loop(0, _NCHUNK, chunk_body, 0, unroll=False)
    for _ in range(_NBUF):
        write_wait()


def kernel(hidden_states, ent_mask):
    B, L, D = hidden_states.shape
    flat = hidden_states.reshape(B * L, D)
    mask_i32 = ent_mask.astype(jnp.int32)

    mesh = plsc.VectorSubcoreMesh(core_axis_name="c", subcore_axis_name="s")
    run = functools.partial(
        pl.kernel,
        mesh=mesh,
        out_type=jax.ShapeDtypeStruct((B * L, D), hidden_states.dtype),
        scratch_types=[
            pltpu.VMEM((_L,), jnp.int32),
            pltpu.VMEM((_RPW,), jnp.int32),
            pltpu.VMEM((_NBUF * _CH, _D), jnp.float32),
            pltpu.SemaphoreType.DMA,
            pltpu.SemaphoreType.DMA,
        ],
        compiler_params=pltpu.CompilerParams(needs_layout_passes=False),
    )(_sc_body)
    out = run(flat, mask_i32)
    return out.reshape(B, L, D)


# speedup vs baseline: 1.0314x; 1.0314x over previous
import functools

import jax
import jax.numpy as jnp
from jax import lax
from jax.experimental import pallas as pl
from jax.experimental.pallas import tpu as pltpu
from jax.experimental.pallas import tpu_sc as plsc

_B, _L, _D = 16, 4096, 1024
_NC, _NS, _LANES = 2, 16, 16
_NW = _NC * _NS
_RPW = _B * _L // _NW
_CH = 8
_NBUF = 6
_LA = 3
_NSP = 3
_NCHUNK = _RPW // _CH


def _sc_body(
    hid_hbm, mask_hbm, out_hbm, mask_v, lidx_v, bufs_v, spbufs, sem_g, sem_h,
    sem_s2,
):
    cid = lax.axis_index("c")
    sid = lax.axis_index("s")
    wid = sid * _NC + cid
    b = wid // 2
    half = wid % 2
    lo = half * _RPW
    row0 = b * _L

    pltpu.sync_copy(mask_hbm.at[b], mask_v)

    def init_body(i, _):
        lidx_v[pl.ds(i * _LANES, _LANES)] = jnp.full((_LANES,), row0, jnp.int32)
        return 0

    lax.fori_loop(0, _RPW // _LANES, init_body, 0, unroll=False)

    def scan_body(i, carry):
        m = mask_v[pl.ds(i * _LANES, _LANES)]
        rank = plsc.cumsum(m) + (carry - 1)
        tok = lax.iota(jnp.int32, _LANES) + (i * _LANES + row0)
        valid = (m > 0) & (rank >= lo) & (rank < lo + _RPW)
        local = jnp.clip(rank - lo, 0, _RPW - 1)
        plsc.store_scatter(lidx_v, [local], tok, mask=valid)
        return carry + jnp.sum(m)

    lax.fori_loop(0, _L // _LANES, scan_body, jnp.int32(0), unroll=False)

    out_base = wid * _RPW
    my_sp = spbufs.at[sid]

    def gather_start(c, p):
        pltpu.async_copy(
            hid_hbm.at[lidx_v.at[pl.ds(c * _CH, _CH)]],
            bufs_v.at[pl.ds(p * _CH, _CH)],
            sem_g,
        )

    def gather_wait():
        pltpu.make_async_copy(
            hid_hbm.at[lidx_v.at[pl.ds(0, _CH)]],
            bufs_v.at[pl.ds(0, _CH)],
            sem_g,
        ).wait()

    def hop1_start(c):
        pltpu.async_copy(
            bufs_v.at[pl.ds((c % _NBUF) * _CH, _CH)],
            my_sp.at[pl.ds((c % _NSP) * _CH, _CH)],
            sem_h,
        )

    def hop1_wait():
        pltpu.make_async_copy(
            bufs_v.at[pl.ds(0, _CH)],
            my_sp.at[pl.ds(0, _CH)],
            sem_h,
        ).wait()

    def hop2_start(c):
        pltpu.async_copy(
            my_sp.at[pl.ds((c % _NSP) * _CH, _CH)],
            out_hbm.at[pl.ds(out_base + c * _CH, _CH)],
            sem_s2,
        )

    def hop2_wait():
        pltpu.make_async_copy(
            my_sp.at[pl.ds(0, _CH)],
            out_hbm.at[pl.ds(out_base, _CH)],
            sem_s2,
        ).wait()

    for c0 in range(_LA):
        gather_start(c0, c0)

    def chunk_body(c, _):
        @pl.when(c >= 1)
        def _():
            hop1_wait()
            hop2_start(c - 1)

        @pl.when(c + _LA < _NCHUNK)
        def _():
            gather_start(c + _LA, (c + _LA) % _NBUF)

        gather_wait()

        @pl.when(c >= _NSP)
        def _():
            hop2_wait()

        hop1_start(c)
        return 0

    lax.fori_loop(0, _NCHUNK, chunk_body, 0, unroll=False)
    hop1_wait()
    hop2_start(_NCHUNK - 1)
    for _ in range(_NSP):
        hop2_wait()


def kernel(hidden_states, ent_mask):
    B, L, D = hidden_states.shape
    flat = hidden_states.reshape(B * L, D)
    mask_i32 = ent_mask.astype(jnp.int32)

    mesh = plsc.VectorSubcoreMesh(core_axis_name="c", subcore_axis_name="s")
    run = functools.partial(
        pl.kernel,
        mesh=mesh,
        out_type=jax.ShapeDtypeStruct((B * L, D), hidden_states.dtype),
        scratch_types=[
            pltpu.VMEM((_L,), jnp.int32),
            pltpu.VMEM((_RPW,), jnp.int32),
            pltpu.VMEM((_NBUF * _CH, _D), jnp.float32),
            pltpu.VMEM_SHARED((_NS, _NSP * _CH, _D), jnp.float32),
            pltpu.SemaphoreType.DMA,
            pltpu.SemaphoreType.DMA,
            pltpu.SemaphoreType.DMA,
        ],
        compiler_params=pltpu.CompilerParams(needs_layout_passes=False),
    )(_sc_body)
    out = run(flat, mask_i32)
    return out.reshape(B, L, D)


# speedup vs baseline: 1.0321x; 1.0006x over previous
import functools

import jax
import jax.numpy as jnp
from jax import lax
from jax.experimental import pallas as pl
from jax.experimental.pallas import tpu as pltpu
from jax.experimental.pallas import tpu_sc as plsc

_B, _L, _D = 16, 4096, 1024
_NC, _NS, _LANES = 2, 16, 16
_NW = _NC * _NS
_RPW = _B * _L // _NW
_CH = 8
_NBUF = 10
_LA = 5
_NSP = 3
_NCHUNK = _RPW // _CH


def _sc_body(
    hid_hbm, mask_hbm, out_hbm, mask_v, lidx_v, bufs_v, spbufs, sem_g, sem_h,
    sem_s2,
):
    cid = lax.axis_index("c")
    sid = lax.axis_index("s")
    wid = sid * _NC + cid
    b = wid // 2
    half = wid % 2
    lo = half * _RPW
    row0 = b * _L

    pltpu.sync_copy(mask_hbm.at[b], mask_v)

    def init_body(i, _):
        lidx_v[pl.ds(i * _LANES, _LANES)] = jnp.full((_LANES,), row0, jnp.int32)
        return 0

    lax.fori_loop(0, _RPW // _LANES, init_body, 0, unroll=False)

    def scan_body(i, carry):
        m = mask_v[pl.ds(i * _LANES, _LANES)]
        rank = plsc.cumsum(m) + (carry - 1)
        tok = lax.iota(jnp.int32, _LANES) + (i * _LANES + row0)
        valid = (m > 0) & (rank >= lo) & (rank < lo + _RPW)
        local = jnp.clip(rank - lo, 0, _RPW - 1)
        plsc.store_scatter(lidx_v, [local], tok, mask=valid)
        return carry + jnp.sum(m)

    lax.fori_loop(0, _L // _LANES, scan_body, jnp.int32(0), unroll=False)

    out_base = wid * _RPW
    my_sp = spbufs.at[sid]

    def gather_start(c, p):
        pltpu.async_copy(
            hid_hbm.at[lidx_v.at[pl.ds(c * _CH, _CH)]],
            bufs_v.at[pl.ds(p * _CH, _CH)],
            sem_g,
        )

    def gather_wait():
        pltpu.make_async_copy(
            hid_hbm.at[lidx_v.at[pl.ds(0, _CH)]],
            bufs_v.at[pl.ds(0, _CH)],
            sem_g,
        ).wait()

    def hop1_start(c):
        pltpu.async_copy(
            bufs_v.at[pl.ds((c % _NBUF) * _CH, _CH)],
            my_sp.at[pl.ds((c % _NSP) * _CH, _CH)],
            sem_h,
        )

    def hop1_wait():
        pltpu.make_async_copy(
            bufs_v.at[pl.ds(0, _CH)],
            my_sp.at[pl.ds(0, _CH)],
            sem_h,
        ).wait()

    def hop2_start(c):
        pltpu.async_copy(
            my_sp.at[pl.ds((c % _NSP) * _CH, _CH)],
            out_hbm.at[pl.ds(out_base + c * _CH, _CH)],
            sem_s2,
        )

    def hop2_wait():
        pltpu.make_async_copy(
            my_sp.at[pl.ds(0, _CH)],
            out_hbm.at[pl.ds(out_base, _CH)],
            sem_s2,
        ).wait()

    for c0 in range(_LA):
        gather_start(c0, c0)

    def chunk_body(c, _):
        @pl.when(c >= 1)
        def _():
            hop1_wait()
            hop2_start(c - 1)

        @pl.when(c + _LA < _NCHUNK)
        def _():
            gather_start(c + _LA, (c + _LA) % _NBUF)

        gather_wait()

        @pl.when(c >= _NSP)
        def _():
            hop2_wait()

        hop1_start(c)
        return 0

    lax.fori_loop(0, _NCHUNK, chunk_body, 0, unroll=False)
    hop1_wait()
    hop2_start(_NCHUNK - 1)
    for _ in range(_NSP):
        hop2_wait()


def kernel(hidden_states, ent_mask):
    B, L, D = hidden_states.shape
    flat = hidden_states.reshape(B * L, D)
    mask_i32 = ent_mask.astype(jnp.int32)

    mesh = plsc.VectorSubcoreMesh(core_axis_name="c", subcore_axis_name="s")
    run = functools.partial(
        pl.kernel,
        mesh=mesh,
        out_type=jax.ShapeDtypeStruct((B * L, D), hidden_states.dtype),
        scratch_types=[
            pltpu.VMEM((_L,), jnp.int32),
            pltpu.VMEM((_RPW,), jnp.int32),
            pltpu.VMEM((_NBUF * _CH, _D), jnp.float32),
            pltpu.VMEM_SHARED((_NS, _NSP * _CH, _D), jnp.float32),
            pltpu.SemaphoreType.DMA,
            pltpu.SemaphoreType.DMA,
            pltpu.SemaphoreType.DMA,
        ],
        compiler_params=pltpu.CompilerParams(needs_layout_passes=False),
    )(_sc_body)
    out = run(flat, mask_i32)
    return out.reshape(B, L, D)
